# baseline (speedup 1.0000x reference)
import functools

import jax
import jax.numpy as jnp
from jax import lax
from jax.experimental import pallas as pl
from jax.experimental.pallas import tpu as pltpu
from jax.experimental.pallas import tpu_sc as plsc

EMB = 64

NUM_CORES = 2
NUM_SUBCORES = 16
NUM_WORKERS = NUM_CORES * NUM_SUBCORES

GATHER = 128
CHUNK = 512
GATHERS_PER_CHUNK = CHUNK // GATHER


@functools.partial(jax.jit, static_argnames=("b_total",))
def _embedding_gather(x2d, table, *, b_total):
    b_per_w = b_total // NUM_WORKERS
    n_chunks = b_per_w // CHUNK
    idx_rows_per_w = b_per_w // GATHER

    mesh = plsc.VectorSubcoreMesh(core_axis_name="c", subcore_axis_name="s")

    @functools.partial(
        pl.kernel,
        out_type=jax.ShapeDtypeStruct((b_total, EMB), jnp.float32),
        mesh=mesh,
        scratch_types=[
            pltpu.VMEM((GATHERS_PER_CHUNK, GATHER), jnp.int32),
            pltpu.VMEM((CHUNK, EMB), jnp.float32),
            pltpu.SemaphoreType.DMA,
        ],
        compiler_params=pltpu.CompilerParams(use_tc_tiling_on_sc=False),
    )
    def k(x_hbm, table_hbm, out_hbm, idx_v, rows_v, sem):
        wid = lax.axis_index("s") * NUM_CORES + lax.axis_index("c")
        out_base = wid * b_per_w
        idx_row_base = wid * idx_rows_per_w

        def body(g, carry):
            pltpu.sync_copy(
                x_hbm.at[pl.ds(idx_row_base + g * GATHERS_PER_CHUNK,
                               GATHERS_PER_CHUNK)],
                idx_v,
            )
            copies = []
            for j in range(GATHERS_PER_CHUNK):
                copies.append(
                    pltpu.async_copy(
                        table_hbm.at[idx_v.at[j]],
                        rows_v.at[pl.ds(j * GATHER, GATHER)],
                        sem,
                    )
                )
            for c in copies:
                c.wait()
            pltpu.sync_copy(
                rows_v,
                out_hbm.at[pl.ds(out_base + g * CHUNK, CHUNK)],
            )
            return carry

        lax.fori_loop(0, n_chunks, body, 0)

    return k(x2d, table)


def kernel(x, table):
    b_total = x.size
    x2d = x.reshape(b_total // GATHER, GATHER).astype(jnp.int32)
    out = _embedding_gather(x2d, table, b_total=b_total)
    return out.reshape(x.shape + (EMB,))


# speedup vs baseline: 1.0457x; 1.0457x over previous
import functools

import jax
import jax.numpy as jnp
from jax import lax
from jax.experimental import pallas as pl
from jax.experimental.pallas import tpu as pltpu
from jax.experimental.pallas import tpu_sc as plsc

EMB = 64

NUM_CORES = 2
NUM_SUBCORES = 16
NUM_WORKERS = NUM_CORES * NUM_SUBCORES

GATHER = 128
CHUNK = 512
GATHERS_PER_CHUNK = CHUNK // GATHER


@functools.partial(jax.jit, static_argnames=("b_total",))
def _embedding_gather(x2d, table, *, b_total):
    b_per_w = b_total // NUM_WORKERS
    n_chunks = b_per_w // CHUNK
    idx_rows_per_w = b_per_w // GATHER

    mesh = plsc.VectorSubcoreMesh(core_axis_name="c", subcore_axis_name="s")

    @functools.partial(
        pl.kernel,
        out_type=jax.ShapeDtypeStruct((b_total, EMB), jnp.float32),
        mesh=mesh,
        scratch_types=[
            pltpu.VMEM((idx_rows_per_w, GATHER), jnp.int32),
            pltpu.VMEM((CHUNK, EMB), jnp.float32),
            pltpu.VMEM((CHUNK, EMB), jnp.float32),
            pltpu.SemaphoreType.DMA,
            pltpu.SemaphoreType.DMA,
            pltpu.SemaphoreType.DMA,
            pltpu.SemaphoreType.DMA,
        ],
        compiler_params=pltpu.CompilerParams(use_tc_tiling_on_sc=False),
    )
    def k(x_hbm, table_hbm, out_hbm, idx_v, rows0, rows1,
          gsem0, gsem1, ssem0, ssem1):
        wid = lax.axis_index("s") * NUM_CORES + lax.axis_index("c")
        out_base = wid * b_per_w
        idx_row_base = wid * idx_rows_per_w

        rows = (rows0, rows1)
        gsems = (gsem0, gsem1)
        ssems = (ssem0, ssem1)

        pltpu.sync_copy(x_hbm.at[pl.ds(idx_row_base, idx_rows_per_w)], idx_v)

        def fire_gathers(g, b):
            for j in range(GATHERS_PER_CHUNK):
                pltpu.async_copy(
                    table_hbm.at[idx_v.at[g * GATHERS_PER_CHUNK + j]],
                    rows[b].at[pl.ds(j * GATHER, GATHER)],
                    gsems[b],
                )

        def wait_gathers(b):
            pltpu.make_async_copy(
                out_hbm.at[pl.ds(out_base, CHUNK)], rows[b], gsems[b]
            ).wait()

        def fire_store(g, b):
            pltpu.async_copy(
                rows[b], out_hbm.at[pl.ds(out_base + g * CHUNK, CHUNK)],
                ssems[b],
            )

        def wait_store(b):
            pltpu.make_async_copy(
                rows[b], out_hbm.at[pl.ds(out_base, CHUNK)], ssems[b]
            ).wait()

        fire_gathers(0, 0)

        def outer(go, carry):
            for b in range(2):
                g = 2 * go + b
                nb = 1 - b

                @pl.when(g >= 1)
                def _():
                    wait_store(nb)

                @pl.when(g + 1 < n_chunks)
                def _():
                    fire_gathers(g + 1, nb)

                wait_gathers(b)
                fire_store(g, b)
            return carry

        lax.fori_loop(0, n_chunks // 2, outer, 0)
        wait_store((n_chunks - 1) % 2)

    return k(x2d, table)


def kernel(x, table):
    b_total = x.size
    x2d = x.reshape(b_total // GATHER, GATHER).astype(jnp.int32)
    out = _embedding_gather(x2d, table, b_total=b_total)
    return out.reshape(x.shape + (EMB,))
